# baseline (speedup 1.0000x reference)
import functools

import jax
import jax.numpy as jnp
from jax.experimental import pallas as pl
from jax.experimental.pallas import tpu as pltpu

_ANCHOR_W = (10.0, 16.0, 33.0)
_ANCHOR_H = (13.0, 30.0, 23.0)
_NUM_ATTRS = 85


def _decode_body(x_ref, o_hbm, scratch, sems, *, gs, stride, bs):
    s = gs * gs
    nA = len(_ANCHOR_W)
    b = pl.program_id(0)
    buf = jax.lax.rem(b, 2)

    k = jax.lax.broadcasted_iota(jnp.int32, (1, s), 1)
    gx = (k % gs).astype(jnp.float32)
    gy = (k // gs).astype(jnp.float32)

    for a in range(nA):
        @pl.when(b >= 2)
        def _wait_prev():
            pltpu.make_async_copy(
                scratch.at[buf, a * s:(a + 1) * s, :],
                o_hbm.at[b - 2, a * s:(a + 1) * s, :],
                sems.at[buf, a],
            ).wait()

        xb = x_ref[0, a * _NUM_ATTRS:(a + 1) * _NUM_ATTRS].reshape(_NUM_ATTRS, s)
        tx = xb[0:1, :]
        ty = xb[1:2, :]
        tw = xb[2:3, :]
        th = xb[3:4, :]
        conf = xb[4:5, :]
        cls = xb[5:, :]

        bx = (jax.nn.sigmoid(tx) + gx) * stride
        by = (jax.nn.sigmoid(ty) + gy) * stride
        bw = jnp.exp(tw) * _ANCHOR_W[a]
        bh = jnp.exp(th) * _ANCHOR_H[a]
        pc = jax.nn.sigmoid(conf)

        m = jnp.max(cls, axis=0, keepdims=True)
        e = jnp.exp(cls - m)
        sm = e / jnp.sum(e, axis=0, keepdims=True)

        res = jnp.concatenate([bx, by, bw, bh, pc, sm], axis=0)
        scratch[buf, a * s:(a + 1) * s, :] = res.T

        pltpu.make_async_copy(
            scratch.at[buf, a * s:(a + 1) * s, :],
            o_hbm.at[b, a * s:(a + 1) * s, :],
            sems.at[buf, a],
        ).start()

    @pl.when(b == bs - 1)
    def _drain():
        for a in range(nA):
            pltpu.make_async_copy(
                scratch.at[1 - buf, a * s:(a + 1) * s, :],
                o_hbm.at[b - 1, a * s:(a + 1) * s, :],
                sems.at[1 - buf, a],
            ).wait()
            pltpu.make_async_copy(
                scratch.at[buf, a * s:(a + 1) * s, :],
                o_hbm.at[b, a * s:(a + 1) * s, :],
                sems.at[buf, a],
            ).wait()


def kernel(x):
    bs, ch, gs, _ = x.shape
    nA = len(_ANCHOR_W)
    s = gs * gs
    stride = 608 // gs
    out = pl.pallas_call(
        functools.partial(_decode_body, gs=gs, stride=float(stride), bs=bs),
        grid=(bs,),
        in_specs=[pl.BlockSpec((1, ch, gs, gs), lambda b: (b, 0, 0, 0))],
        out_specs=pl.BlockSpec(memory_space=pltpu.MemorySpace.HBM),
        out_shape=jax.ShapeDtypeStruct((bs, nA * s, _NUM_ATTRS), jnp.float32),
        scratch_shapes=[
            pltpu.VMEM((2, nA * s, _NUM_ATTRS), jnp.float32),
            pltpu.SemaphoreType.DMA((2, nA)),
        ],
    )(x)
    return out


# speedup vs baseline: 1.0929x; 1.0929x over previous
import functools

import jax
import jax.numpy as jnp
from jax.experimental import pallas as pl

_ANCHOR_W = (10.0, 16.0, 33.0)
_ANCHOR_H = (13.0, 30.0, 23.0)
_NUM_ATTRS = 85


def _decode_body(x_ref, o_ref, *, gs, stride):
    s = gs * gs
    k = jax.lax.broadcasted_iota(jnp.int32, (1, s), 1)
    gx = (k % gs).astype(jnp.float32)
    gy = (k // gs).astype(jnp.float32)
    for a in range(len(_ANCHOR_W)):
        xb = x_ref[0, a * _NUM_ATTRS:(a + 1) * _NUM_ATTRS].reshape(_NUM_ATTRS, s)
        tx = xb[0:1, :]
        ty = xb[1:2, :]
        tw = xb[2:3, :]
        th = xb[3:4, :]
        conf = xb[4:5, :]
        cls = xb[5:, :]

        bx = (jax.nn.sigmoid(tx) + gx) * stride
        by = (jax.nn.sigmoid(ty) + gy) * stride
        bw = jnp.exp(tw) * _ANCHOR_W[a]
        bh = jnp.exp(th) * _ANCHOR_H[a]
        pc = jax.nn.sigmoid(conf)

        m = jnp.max(cls, axis=0, keepdims=True)
        e = jnp.exp(cls - m)
        sm = e / jnp.sum(e, axis=0, keepdims=True)

        res = jnp.concatenate([bx, by, bw, bh, pc, sm], axis=0)
        o_ref[0, a] = res.T


def kernel(x):
    bs, ch, gs, _ = x.shape
    nA = len(_ANCHOR_W)
    s = gs * gs
    stride = 608 // gs
    out = pl.pallas_call(
        functools.partial(_decode_body, gs=gs, stride=float(stride)),
        grid=(bs,),
        in_specs=[pl.BlockSpec((1, ch, gs, gs), lambda b: (b, 0, 0, 0))],
        out_specs=pl.BlockSpec((1, nA, s, _NUM_ATTRS), lambda b: (b, 0, 0, 0)),
        out_shape=jax.ShapeDtypeStruct((bs, nA, s, _NUM_ATTRS), jnp.float32),
    )(x)
    return out.reshape(bs, nA * s, _NUM_ATTRS)
